# baseline (speedup 1.0000x reference)
import jax
import jax.numpy as jnp
from jax.experimental import pallas as pl
from jax.experimental.pallas import tpu as pltpu

B = 128
K = 8
LOCAL_E = 8
H = 2048
I = 1024
IT = 256
NI = I // IT


def _ffn_body(ids_ref, scl_ref, act_ref, x_ref, smooth_ref,
              w1g_ref, w1u_ref, s1g_ref, s1u_ref, w2_ref, s2_ref,
              out_ref):
    e = pl.program_id(0)
    i = pl.program_id(1)

    m = (ids_ref[...] == e).astype(jnp.float32)
    w_col = jnp.sum(m * scl_ref[...], axis=1, keepdims=True) * act_ref[...]

    xs = x_ref[...] * smooth_ref[0]
    g = jnp.dot(xs, w1g_ref[0], preferred_element_type=jnp.float32,
                precision=jax.lax.Precision.HIGHEST) * s1g_ref[0]
    u = jnp.dot(xs, w1u_ref[0], preferred_element_type=jnp.float32,
                precision=jax.lax.Precision.HIGHEST) * s1u_ref[0]
    a = (g * jax.nn.sigmoid(g)) * u
    part = jnp.dot(a, w2_ref[0], preferred_element_type=jnp.float32,
                   precision=jax.lax.Precision.HIGHEST)

    @pl.when((e == 0) & (i == 0))
    def _():
        out_ref[...] = jnp.zeros_like(out_ref)

    out_ref[...] += part * s2_ref[0] * w_col


def kernel(x, expert_ids, smooth_scales, expert_scales, x_active_mask,
           gmm1_weight, gmm1_weight_scale, gmm2_weight, gmm2_weight_scale):
    act_col = x_active_mask.astype(jnp.float32).reshape(B, 1)
    smooth3 = smooth_scales.reshape(LOCAL_E, 1, H)
    s1_3 = gmm1_weight_scale.reshape(LOCAL_E, 1, 2 * I)
    s2_3 = gmm2_weight_scale.reshape(LOCAL_E, 1, H)

    grid = (LOCAL_E, NI)
    out = pl.pallas_call(
        _ffn_body,
        grid=grid,
        in_specs=[
            pl.BlockSpec((B, K), lambda e, i: (0, 0)),
            pl.BlockSpec((B, K), lambda e, i: (0, 0)),
            pl.BlockSpec((B, 1), lambda e, i: (0, 0)),
            pl.BlockSpec((B, H), lambda e, i: (0, 0)),
            pl.BlockSpec((1, 1, H), lambda e, i: (e, 0, 0)),
            pl.BlockSpec((1, H, IT), lambda e, i: (e, 0, i)),
            pl.BlockSpec((1, H, IT), lambda e, i: (e, 0, NI + i)),
            pl.BlockSpec((1, 1, IT), lambda e, i: (e, 0, i)),
            pl.BlockSpec((1, 1, IT), lambda e, i: (e, 0, NI + i)),
            pl.BlockSpec((1, IT, H), lambda e, i: (e, i, 0)),
            pl.BlockSpec((1, 1, H), lambda e, i: (e, 0, 0)),
        ],
        out_specs=pl.BlockSpec((B, H), lambda e, i: (0, 0)),
        out_shape=jax.ShapeDtypeStruct((B, H), jnp.float32),
        compiler_params=pltpu.CompilerParams(
            dimension_semantics=("arbitrary", "arbitrary"),
        ),
    )(expert_ids, expert_scales, act_col, x, smooth3,
      gmm1_weight, gmm1_weight, s1_3, s1_3,
      gmm2_weight, s2_3)
    return out


# speedup vs baseline: 1.6348x; 1.6348x over previous
import jax
import jax.numpy as jnp
from jax.experimental import pallas as pl
from jax.experimental.pallas import tpu as pltpu

B = 128
K = 8
LOCAL_E = 8
H = 2048
I = 1024
IT = 256
NI = I // IT


def _ffn_body(ids_ref, scl_ref, act_ref, x_ref, smooth_ref,
              w1g_ref, w1u_ref, s1g_ref, s1u_ref, w2_ref, s2_ref,
              out_ref):
    e = pl.program_id(0)
    i = pl.program_id(1)

    m = (ids_ref[...] == e).astype(jnp.float32)
    w_col = jnp.sum(m * scl_ref[...], axis=1, keepdims=True) * act_ref[...]

    xs = x_ref[...] * smooth_ref[0]
    g = jnp.dot(xs, w1g_ref[0], preferred_element_type=jnp.float32) * s1g_ref[0]
    u = jnp.dot(xs, w1u_ref[0], preferred_element_type=jnp.float32) * s1u_ref[0]
    a = (g * jax.nn.sigmoid(g)) * u
    part = jnp.dot(a, w2_ref[0], preferred_element_type=jnp.float32)

    @pl.when((e == 0) & (i == 0))
    def _():
        out_ref[...] = jnp.zeros_like(out_ref)

    out_ref[...] += part * s2_ref[0] * w_col


def kernel(x, expert_ids, smooth_scales, expert_scales, x_active_mask,
           gmm1_weight, gmm1_weight_scale, gmm2_weight, gmm2_weight_scale):
    act_col = x_active_mask.astype(jnp.float32).reshape(B, 1)
    smooth3 = smooth_scales.reshape(LOCAL_E, 1, H)
    s1_3 = gmm1_weight_scale.reshape(LOCAL_E, 1, 2 * I)
    s2_3 = gmm2_weight_scale.reshape(LOCAL_E, 1, H)

    grid = (LOCAL_E, NI)
    out = pl.pallas_call(
        _ffn_body,
        grid=grid,
        in_specs=[
            pl.BlockSpec((B, K), lambda e, i: (0, 0)),
            pl.BlockSpec((B, K), lambda e, i: (0, 0)),
            pl.BlockSpec((B, 1), lambda e, i: (0, 0)),
            pl.BlockSpec((B, H), lambda e, i: (0, 0)),
            pl.BlockSpec((1, 1, H), lambda e, i: (e, 0, 0)),
            pl.BlockSpec((1, H, IT), lambda e, i: (e, 0, i)),
            pl.BlockSpec((1, H, IT), lambda e, i: (e, 0, NI + i)),
            pl.BlockSpec((1, 1, IT), lambda e, i: (e, 0, i)),
            pl.BlockSpec((1, 1, IT), lambda e, i: (e, 0, NI + i)),
            pl.BlockSpec((1, IT, H), lambda e, i: (e, i, 0)),
            pl.BlockSpec((1, 1, H), lambda e, i: (e, 0, 0)),
        ],
        out_specs=pl.BlockSpec((B, H), lambda e, i: (0, 0)),
        out_shape=jax.ShapeDtypeStruct((B, H), jnp.float32),
        compiler_params=pltpu.CompilerParams(
            dimension_semantics=("arbitrary", "arbitrary"),
        ),
    )(expert_ids, expert_scales, act_col, x, smooth3,
      gmm1_weight, gmm1_weight, s1_3, s1_3,
      gmm2_weight, s2_3)
    return out


# speedup vs baseline: 1.7044x; 1.0426x over previous
import jax
import jax.numpy as jnp
from jax.experimental import pallas as pl
from jax.experimental.pallas import tpu as pltpu

B = 128
K = 8
LOCAL_E = 8
H = 2048
I = 1024
IT = 512
NI = I // IT


def _ffn_body(ids_ref, scl_ref, act_ref, x_ref, smooth_ref,
              w1g_ref, w1u_ref, s1g_ref, s1u_ref, w2_ref, s2_ref,
              out_ref):
    e = pl.program_id(0)
    i = pl.program_id(1)

    m = (ids_ref[...] == e).astype(jnp.float32)
    w_col = jnp.sum(m * scl_ref[...], axis=1, keepdims=True) * act_ref[...]

    xs = x_ref[...] * smooth_ref[0]
    g = jnp.dot(xs, w1g_ref[0], preferred_element_type=jnp.float32) * s1g_ref[0]
    u = jnp.dot(xs, w1u_ref[0], preferred_element_type=jnp.float32) * s1u_ref[0]
    a = (g * jax.nn.sigmoid(g)) * u
    part = jnp.dot(a, w2_ref[0], preferred_element_type=jnp.float32)

    @pl.when((e == 0) & (i == 0))
    def _():
        out_ref[...] = jnp.zeros_like(out_ref)

    out_ref[...] += part * s2_ref[0] * w_col


def kernel(x, expert_ids, smooth_scales, expert_scales, x_active_mask,
           gmm1_weight, gmm1_weight_scale, gmm2_weight, gmm2_weight_scale):
    act_col = x_active_mask.astype(jnp.float32).reshape(B, 1)
    smooth3 = smooth_scales.reshape(LOCAL_E, 1, H)
    s1_3 = gmm1_weight_scale.reshape(LOCAL_E, 1, 2 * I)
    s2_3 = gmm2_weight_scale.reshape(LOCAL_E, 1, H)

    grid = (LOCAL_E, NI)
    out = pl.pallas_call(
        _ffn_body,
        grid=grid,
        in_specs=[
            pl.BlockSpec((B, K), lambda e, i: (0, 0)),
            pl.BlockSpec((B, K), lambda e, i: (0, 0)),
            pl.BlockSpec((B, 1), lambda e, i: (0, 0)),
            pl.BlockSpec((B, H), lambda e, i: (0, 0)),
            pl.BlockSpec((1, 1, H), lambda e, i: (e, 0, 0)),
            pl.BlockSpec((1, H, IT), lambda e, i: (e, 0, i)),
            pl.BlockSpec((1, H, IT), lambda e, i: (e, 0, NI + i)),
            pl.BlockSpec((1, 1, IT), lambda e, i: (e, 0, i)),
            pl.BlockSpec((1, 1, IT), lambda e, i: (e, 0, NI + i)),
            pl.BlockSpec((1, IT, H), lambda e, i: (e, i, 0)),
            pl.BlockSpec((1, 1, H), lambda e, i: (e, 0, 0)),
        ],
        out_specs=pl.BlockSpec((B, H), lambda e, i: (0, 0)),
        out_shape=jax.ShapeDtypeStruct((B, H), jnp.float32),
        compiler_params=pltpu.CompilerParams(
            dimension_semantics=("arbitrary", "arbitrary"),
        ),
    )(expert_ids, expert_scales, act_col, x, smooth3,
      gmm1_weight, gmm1_weight, s1_3, s1_3,
      gmm2_weight, s2_3)
    return out
